# Initial kernel scaffold; baseline (speedup 1.0000x reference)
import jax
import jax.numpy as jnp
from jax.experimental import pallas as pl


def kernel(hidden_states, router_weight, correction_bias, gate_up_proj, down_proj, shared_gate, shared_up, shared_down):
    raise NotImplementedError("write your pallas kernel here")


# speedup vs baseline: 1.5806x; 1.5806x over previous
import functools

import jax
import jax.numpy as jnp
from jax import lax
from jax.experimental import pallas as pl
from jax.experimental.pallas import tpu as pltpu

B, S, D = 2, 2048, 1024
E, K, F = 8, 2, 1024
I = 4096
LIMIT = 7.0
RSF = 2.5

N = B * S
RT = 512
NRT = N // RT
IC = 1024
NIC = I // IC


def _router_body(x_ref, rw_ref, cb_ref, w8_ref):
    x = x_ref[...]
    logits = lax.dot_general(x, rw_ref[...], (((1,), (1,)), ((), ())),
                             preferred_element_type=jnp.float32)
    scores = jax.nn.sigmoid(logits)
    biased = scores + cb_ref[...]
    eidx = lax.broadcasted_iota(jnp.int32, (RT, E), 1)
    m1 = jnp.max(biased, axis=1, keepdims=True)
    i1 = jnp.min(jnp.where(biased == m1, eidx, E), axis=1, keepdims=True)
    sel1 = eidx == i1
    b2 = jnp.where(sel1, -jnp.inf, biased)
    m2 = jnp.max(b2, axis=1, keepdims=True)
    i2 = jnp.min(jnp.where(b2 == m2, eidx, E), axis=1, keepdims=True)
    sel2 = eidx == i2
    s1 = jnp.sum(jnp.where(sel1, scores, 0.0), axis=1, keepdims=True)
    s2 = jnp.sum(jnp.where(sel2, scores, 0.0), axis=1, keepdims=True)
    denom = s1 + s2 + 1e-20
    w8 = jnp.where(sel1, s1, 0.0) + jnp.where(sel2, s2, 0.0)
    w8_ref[...] = w8 * (RSF / denom)


def _expert_body(x_ref, gu_ref, dn_ref, w8_ref, out_ref):
    e = pl.program_id(1)
    x = x_ref[...]
    gu = lax.dot_general(x, gu_ref[0], (((1,), (1,)), ((), ())),
                         preferred_element_type=jnp.float32)
    gate = jnp.minimum(gu[:, :F], LIMIT)
    up = jnp.clip(gu[:, F:], -LIMIT, LIMIT)
    act = gate * jax.nn.sigmoid(gate) * up
    cur = lax.dot_general(act, dn_ref[0], (((1,), (1,)), ((), ())),
                          preferred_element_type=jnp.float32)
    onehot = (lax.broadcasted_iota(jnp.int32, (RT, E), 1) == e)
    w = jnp.sum(jnp.where(onehot, w8_ref[...], 0.0), axis=1, keepdims=True)
    contrib = cur * w

    @pl.when(e == 0)
    def _():
        out_ref[...] = contrib

    @pl.when(e != 0)
    def _():
        out_ref[...] += contrib


def _shared_body(x_ref, sg_ref, su_ref, sd_ref, routed_ref, out_ref):
    ic = pl.program_id(1)
    x = x_ref[...]
    g = lax.dot_general(x, sg_ref[...], (((1,), (1,)), ((), ())),
                        preferred_element_type=jnp.float32)
    u = lax.dot_general(x, su_ref[...], (((1,), (1,)), ((), ())),
                        preferred_element_type=jnp.float32)
    h = g * jax.nn.sigmoid(g) * u
    part = lax.dot_general(h, sd_ref[...], (((1,), (1,)), ((), ())),
                           preferred_element_type=jnp.float32)

    @pl.when(ic == 0)
    def _():
        out_ref[...] = part + routed_ref[...]

    @pl.when(ic != 0)
    def _():
        out_ref[...] += part


def kernel(hidden_states, router_weight, correction_bias, gate_up_proj,
           down_proj, shared_gate, shared_up, shared_down):
    flat = hidden_states.reshape(N, D)
    cb = correction_bias.reshape(1, E)

    w8 = pl.pallas_call(
        _router_body,
        grid=(NRT,),
        in_specs=[
            pl.BlockSpec((RT, D), lambda r: (r, 0)),
            pl.BlockSpec((E, D), lambda r: (0, 0)),
            pl.BlockSpec((1, E), lambda r: (0, 0)),
        ],
        out_specs=pl.BlockSpec((RT, E), lambda r: (r, 0)),
        out_shape=jax.ShapeDtypeStruct((N, E), jnp.float32),
    )(flat, router_weight, cb)

    routed = pl.pallas_call(
        _expert_body,
        grid=(NRT, E),
        in_specs=[
            pl.BlockSpec((RT, D), lambda r, e: (r, 0)),
            pl.BlockSpec((1, 2 * F, D), lambda r, e: (e, 0, 0)),
            pl.BlockSpec((1, D, F), lambda r, e: (e, 0, 0)),
            pl.BlockSpec((RT, E), lambda r, e: (r, 0)),
        ],
        out_specs=pl.BlockSpec((RT, D), lambda r, e: (r, 0)),
        out_shape=jax.ShapeDtypeStruct((N, D), jnp.float32),
    )(flat, gate_up_proj, down_proj, w8)

    out = pl.pallas_call(
        _shared_body,
        grid=(NRT, NIC),
        in_specs=[
            pl.BlockSpec((RT, D), lambda r, c: (r, 0)),
            pl.BlockSpec((IC, D), lambda r, c: (c, 0)),
            pl.BlockSpec((IC, D), lambda r, c: (c, 0)),
            pl.BlockSpec((D, IC), lambda r, c: (0, c)),
            pl.BlockSpec((RT, D), lambda r, c: (r, 0)),
        ],
        out_specs=pl.BlockSpec((RT, D), lambda r, c: (r, 0)),
        out_shape=jax.ShapeDtypeStruct((N, D), jnp.float32),
    )(flat, shared_gate, shared_up, shared_down, routed)

    return out.reshape(B, S, D)
